# baseline (speedup 1.0000x reference)
import functools

import jax
import jax.numpy as jnp
from jax import lax
from jax.experimental import pallas as pl
from jax.experimental.pallas import tpu as pltpu
from jax.experimental.pallas import tpu_sc as plsc

N = 10000
E = 320000
D = 128

NC = 2
NS = 16
NW = NC * NS
FPW = D // NW

C = 4000
NCHUNK = E // C

DUMMY = FPW * N
LSZ = FPW * N + 16

_NEG_INF = float("-inf")

_GDN = lax.GatherDimensionNumbers(
    offset_dims=(), collapsed_slice_dims=(0,), start_index_map=(0,))


def _perm(x, idx):
    return lax.gather(x, idx.reshape(16, 1), _GDN, (1,),
                      mode=lax.GatherScatterMode.PROMISE_IN_BOUNDS)


def _segmax_sc(hTf, edges):
    mesh = plsc.VectorSubcoreMesh(core_axis_name="c", subcore_axis_name="s",
                                  num_cores=NC, num_subcores=NS)

    @functools.partial(
        pl.kernel,
        out_type=jax.ShapeDtypeStruct((D * N,), jnp.float32),
        mesh=mesh,
        compiler_params=pltpu.CompilerParams(needs_layout_passes=False),
        scratch_types=[
            pltpu.VMEM((LSZ,), jnp.float32),
            pltpu.VMEM((LSZ,), jnp.float32),
            pltpu.VMEM((C,), jnp.int32),
            pltpu.VMEM((C,), jnp.int32),
            pltpu.SemaphoreType.DMA,
            pltpu.SemaphoreType.DMA,
        ],
    )
    def seg_kernel(h_hbm, edges_hbm, out_hbm,
                   hloc, acc, eb0, eb1, esem0, esem1):
        wid = lax.axis_index("s") * NC + lax.axis_index("c")

        eb = (eb0, eb1)
        esem = (esem0, esem1)

        for i in range(FPW):
            pltpu.sync_copy(h_hbm.at[pl.ds((wid * FPW + i) * N, N)],
                            hloc.at[pl.ds(i * N, N)])

        neg = jnp.full((16,), _NEG_INF, jnp.float32)

        @pl.loop(0, LSZ, step=16)
        def _init(r):
            acc[pl.ds(r, 16)] = neg

        def fire_edges(ci, p):
            pltpu.async_copy(edges_hbm.at[pl.ds(ci * C, C)], eb[p], esem[p])

        def drain_edges(p):
            pltpu.make_async_copy(edges_hbm.at[pl.ds(0, C)], eb[p],
                                  esem[p]).wait()

        lane = lax.iota(jnp.int32, 16)
        shift_idx = [jnp.maximum(lane - (1 << s), 0) for s in range(4)]
        next_idx = jnp.minimum(lane + 1, 15)
        is_last_lane = lane == 15

        def process(p):
            ebp = eb[p]

            GB = 4

            def sorted_group(j):
                ev = ebp[pl.ds(j * 16, 16)]
                sev = plsc.sort_key_val(ev, ev)[0]
                sk = lax.shift_right_logical(sev, 16)
                sv = sev & jnp.int32(0xFFFF)
                nxt = _perm(sk, next_idx)
                boundary = (sk != nxt) | is_last_lane
                return sk, sv, boundary

            @pl.loop(0, C // (16 * GB), unroll=2)
            def body(i):
                dup = None
                for t in range(GB):
                    sk, sv, boundary = sorted_group(i * GB + t)
                    d = ~boundary
                    dup = d if dup is None else (dup | d)
                    for f in range(FPW):
                        x = plsc.load_gather(hloc, [sv + f * N])
                        tgt = sk + f * N
                        cur = plsc.load_gather(acc, [tgt])
                        plsc.store_scatter(acc, [tgt], jnp.maximum(cur, x))

                ndup = plsc.all_reduce_population_count(dup)

                @pl.when(jnp.max(ndup) > 0)
                def _dups():
                    for t in range(GB):
                        sk, sv, boundary = sorted_group(i * GB + t)
                        ems = [_perm(sk, ix) == sk for ix in shift_idx]
                        for f in range(FPW):
                            x = plsc.load_gather(hloc, [sv + f * N])
                            for s in range(4):
                                px = _perm(x, shift_idx[s])
                                x = jnp.where(ems[s], jnp.maximum(x, px), x)
                            tgt = jnp.where(boundary, sk + f * N, DUMMY)
                            cur = plsc.load_gather(acc, [tgt])
                            plsc.store_scatter(acc, [tgt],
                                               jnp.maximum(cur, x))

        fire_edges(0, 0)

        def chunk_work(p, ci):
            @pl.when(ci + 1 < NCHUNK)
            def _pf():
                fire_edges(ci + 1, 1 - p)

            drain_edges(p)
            process(p)

        @pl.loop(0, NCHUNK // 2)
        def pair_body(i):
            chunk_work(0, 2 * i)
            chunk_work(1, 2 * i + 1)

        @pl.loop(0, FPW * N, step=16)
        def _fix(r):
            v = acc[pl.ds(r, 16)]
            acc[pl.ds(r, 16)] = jnp.where(v == _NEG_INF, jnp.float32(0.0), v)

        for i in range(FPW):
            pltpu.sync_copy(acc.at[pl.ds(i * N, N)],
                            out_hbm.at[pl.ds((wid * FPW + i) * N, N)])

    return seg_kernel(hTf, edges)


def _linear_tc(agg, h, WlT, WrT, b2d, relu):
    BN = 2000
    grid = (N // BN,)

    def body(a_ref, h_ref, wl_ref, wr_ref, b_ref, o_ref):
        r = jnp.dot(a_ref[...], wl_ref[...],
                    preferred_element_type=jnp.float32)
        r = r + jnp.dot(h_ref[...], wr_ref[...],
                        preferred_element_type=jnp.float32)
        r = r + b_ref[...]
        if relu:
            r = jnp.maximum(r, 0.0)
        o_ref[...] = r

    return pl.pallas_call(
        body,
        grid=grid,
        in_specs=[
            pl.BlockSpec((BN, D), lambda i: (i, 0)),
            pl.BlockSpec((BN, D), lambda i: (i, 0)),
            pl.BlockSpec((D, D), lambda i: (0, 0)),
            pl.BlockSpec((D, D), lambda i: (0, 0)),
            pl.BlockSpec((1, D), lambda i: (0, 0)),
        ],
        out_specs=pl.BlockSpec((BN, D), lambda i: (i, 0)),
        out_shape=jax.ShapeDtypeStruct((N, D), jnp.float32),
    )(agg, h, WlT, WrT, b2d)


def kernel(x, edge_index, W1l, b1l, W1r, W2l, b2l, W2r):
    src = edge_index[0]
    dst = edge_index[1]
    edges = (dst << 16) | src

    agg1 = _segmax_sc(x.T.reshape(-1), edges).reshape(D, N).T
    h1 = _linear_tc(agg1, x, W1l.T, W1r.T, b1l.reshape(1, D), relu=True)
    agg2 = _segmax_sc(h1.T.reshape(-1), edges).reshape(D, N).T
    out = _linear_tc(agg2, h1, W2l.T, W2r.T, b2l.reshape(1, D), relu=False)
    return out.reshape(-1)


# speedup vs baseline: 1.0543x; 1.0543x over previous
import functools

import jax
import jax.numpy as jnp
from jax import lax
from jax.experimental import pallas as pl
from jax.experimental.pallas import tpu as pltpu
from jax.experimental.pallas import tpu_sc as plsc

N = 10000
E = 320000
D = 128

NC = 2
NS = 16
NW = NC * NS
FPW = D // NW

C = 4000
NCHUNK = E // C

DUMMY = FPW * N
LSZ = FPW * N + 16

_NEG_INF = float("-inf")

_GDN = lax.GatherDimensionNumbers(
    offset_dims=(), collapsed_slice_dims=(0,), start_index_map=(0,))


def _perm(x, idx):
    return lax.gather(x, idx.reshape(16, 1), _GDN, (1,),
                      mode=lax.GatherScatterMode.PROMISE_IN_BOUNDS)


def _segmax_sc(hTf, edges):
    mesh = plsc.VectorSubcoreMesh(core_axis_name="c", subcore_axis_name="s",
                                  num_cores=NC, num_subcores=NS)

    @functools.partial(
        pl.kernel,
        out_type=jax.ShapeDtypeStruct((D * N,), jnp.float32),
        mesh=mesh,
        compiler_params=pltpu.CompilerParams(needs_layout_passes=False),
        scratch_types=[
            pltpu.VMEM((LSZ,), jnp.float32),
            pltpu.VMEM((LSZ,), jnp.float32),
            pltpu.VMEM((C,), jnp.int32),
            pltpu.VMEM((C,), jnp.int32),
            pltpu.SemaphoreType.DMA,
            pltpu.SemaphoreType.DMA,
        ],
    )
    def seg_kernel(h_hbm, edges_hbm, out_hbm,
                   hloc, acc, eb0, eb1, esem0, esem1):
        wid = lax.axis_index("s") * NC + lax.axis_index("c")

        eb = (eb0, eb1)
        esem = (esem0, esem1)

        for i in range(FPW):
            pltpu.sync_copy(h_hbm.at[pl.ds((wid * FPW + i) * N, N)],
                            hloc.at[pl.ds(i * N, N)])

        neg = jnp.full((16,), _NEG_INF, jnp.float32)

        @pl.loop(0, LSZ, step=16)
        def _init(r):
            acc[pl.ds(r, 16)] = neg

        def fire_edges(ci, p):
            pltpu.async_copy(edges_hbm.at[pl.ds(ci * C, C)], eb[p], esem[p])

        def drain_edges(p):
            pltpu.make_async_copy(edges_hbm.at[pl.ds(0, C)], eb[p],
                                  esem[p]).wait()

        lane = lax.iota(jnp.int32, 16)
        shift_idx = [jnp.maximum(lane - (1 << s), 0) for s in range(4)]
        next_idx = jnp.minimum(lane + 1, 15)
        is_last_lane = lane == 15

        def process(p):
            ebp = eb[p]

            @pl.loop(0, C // 16, unroll=4)
            def body(i):
                ev = ebp[pl.ds(i * 16, 16)]
                sev = plsc.sort_key_val(ev, ev)[0]
                sk = lax.shift_right_logical(sev, 16)
                sv = sev & jnp.int32(0xFFFF)
                nxt = _perm(sk, next_idx)
                boundary = (sk != nxt) | is_last_lane
                nuniq = plsc.all_reduce_population_count(boundary)

                for f in range(FPW):
                    x = plsc.load_gather(hloc, [sv + f * N])
                    tgt = sk + f * N
                    cur = plsc.load_gather(acc, [tgt])
                    plsc.store_scatter(acc, [tgt], jnp.maximum(cur, x))

                @pl.when(jnp.max(nuniq) < 16)
                def _dups():
                    ems = [_perm(sk, ix) == sk for ix in shift_idx]
                    for f in range(FPW):
                        x = plsc.load_gather(hloc, [sv + f * N])
                        for s in range(4):
                            px = _perm(x, shift_idx[s])
                            x = jnp.where(ems[s], jnp.maximum(x, px), x)
                        tgt = jnp.where(boundary, sk + f * N, DUMMY)
                        cur = plsc.load_gather(acc, [tgt])
                        plsc.store_scatter(acc, [tgt], jnp.maximum(cur, x))

        fire_edges(0, 0)

        def chunk_work(p, ci):
            @pl.when(ci + 1 < NCHUNK)
            def _pf():
                fire_edges(ci + 1, 1 - p)

            drain_edges(p)
            process(p)

        @pl.loop(0, NCHUNK // 2)
        def pair_body(i):
            chunk_work(0, 2 * i)
            chunk_work(1, 2 * i + 1)

        @pl.loop(0, FPW * N, step=16)
        def _fix(r):
            v = acc[pl.ds(r, 16)]
            acc[pl.ds(r, 16)] = jnp.where(v == _NEG_INF, jnp.float32(0.0), v)

        for i in range(FPW):
            pltpu.sync_copy(acc.at[pl.ds(i * N, N)],
                            out_hbm.at[pl.ds((wid * FPW + i) * N, N)])

    return seg_kernel(hTf, edges)


def _linear_tc(agg, h, WlT, WrT, b2d, relu):
    BN = 2000
    grid = (N // BN,)

    def body(a_ref, h_ref, wl_ref, wr_ref, b_ref, o_ref):
        r = jnp.dot(a_ref[...], wl_ref[...],
                    preferred_element_type=jnp.float32)
        r = r + jnp.dot(h_ref[...], wr_ref[...],
                        preferred_element_type=jnp.float32)
        r = r + b_ref[...]
        if relu:
            r = jnp.maximum(r, 0.0)
        o_ref[...] = r

    return pl.pallas_call(
        body,
        grid=grid,
        in_specs=[
            pl.BlockSpec((BN, D), lambda i: (i, 0)),
            pl.BlockSpec((BN, D), lambda i: (i, 0)),
            pl.BlockSpec((D, D), lambda i: (0, 0)),
            pl.BlockSpec((D, D), lambda i: (0, 0)),
            pl.BlockSpec((1, D), lambda i: (0, 0)),
        ],
        out_specs=pl.BlockSpec((BN, D), lambda i: (i, 0)),
        out_shape=jax.ShapeDtypeStruct((N, D), jnp.float32),
    )(agg, h, WlT, WrT, b2d)


def kernel(x, edge_index, W1l, b1l, W1r, W2l, b2l, W2r):
    src = edge_index[0]
    dst = edge_index[1]
    edges = (dst << 16) | src

    agg1 = _segmax_sc(x.T.reshape(-1), edges).reshape(D, N).T
    h1 = _linear_tc(agg1, x, W1l.T, W1r.T, b1l.reshape(1, D), relu=True)
    agg2 = _segmax_sc(h1.T.reshape(-1), edges).reshape(D, N).T
    out = _linear_tc(agg2, h1, W2l.T, W2r.T, b2l.reshape(1, D), relu=False)
    return out.reshape(-1)
